# baseline (speedup 1.0000x reference)
import functools
import math

import jax
import jax.numpy as jnp
import numpy as np
from jax import lax
from jax.experimental import pallas as pl
from jax.experimental.pallas import tpu as pltpu
from jax.experimental.pallas import tpu_sc as plsc

VOCAB = 1_000_000
EMBED = 64
MAX_LEN = 512
SEQ = 200
BATCH = 4096

NC = 2
NS = 16
NW = NC * NS

ROWS = SEQ * BATCH
B_PER_W = ROWS // NW
CH = 512
NCH = B_PER_W // CH
NVEC = EMBED // 16


def _build_pe_np() -> np.ndarray:
    position = np.arange(0, MAX_LEN, dtype=np.float32)[:, None]
    div_term = np.exp(
        np.arange(0, EMBED, 2, dtype=np.float32) * -(math.log(10000.0) / EMBED)
    )
    pe = np.zeros((MAX_LEN, EMBED), dtype=np.float32)
    pe[:, 0::2] = np.sin(position * div_term)
    pe[:, 1::2] = np.cos(position * div_term)
    return pe[:SEQ]


_PE = _build_pe_np()


def _sc_body(table_hbm, x_hbm, pe_hbm, out_hbm,
             idx_v, rows_v, pe_v, gsem0, gsem1, wsem0, wsem1):
    gsems = (gsem0, gsem1)
    wsems = (wsem0, wsem1)

    wid = lax.axis_index("s") * NC + lax.axis_index("c")
    base = pl.multiple_of(wid * B_PER_W, B_PER_W)

    pltpu.sync_copy(x_hbm.at[pl.ds(base, B_PER_W)], idx_v)
    pltpu.sync_copy(pe_hbm, pe_v)

    def g_copy(c, b):
        start = pl.multiple_of(c * CH, CH)
        return pltpu.make_async_copy(
            table_hbm.at[idx_v.at[pl.ds(start, CH)]], rows_v.at[b], gsems[b]
        )

    def w_copy(c, b):
        start = pl.multiple_of(base + c * CH, CH)
        return pltpu.make_async_copy(
            rows_v.at[b], out_hbm.at[pl.ds(start, CH)], wsems[b]
        )

    g_copy(0, 0).start()
    g_copy(1, 1).start()

    def step(t, carry):
        for b in range(2):
            c = 2 * t + b
            g_copy(c, b).wait()

            s = (base + c * CH) // BATCH
            pes = [pe_v[s, pl.ds(k * 16, 16)] for k in range(NVEC)]

            def add_row(r, _, b=b, pes=pes):
                for k in range(NVEC):
                    plsc.addupdate(rows_v.at[b, r, pl.ds(k * 16, 16)], pes[k])
                return _

            lax.fori_loop(0, CH, add_row, 0, unroll=2)

            w_copy(c, b).start()

            @pl.when(c + 2 < NCH)
            def _(c=c, b=b):
                w_copy(c, b).wait()
                g_copy(c + 2, b).start()
        return carry

    lax.fori_loop(0, NCH // 2, step, 0)

    w_copy(NCH - 2, 0).wait()
    w_copy(NCH - 1, 1).wait()


@functools.partial(
    pl.kernel,
    out_type=jax.ShapeDtypeStruct((ROWS, EMBED), jnp.float32),
    mesh=plsc.VectorSubcoreMesh(core_axis_name="c", subcore_axis_name="s"),
    compiler_params=pltpu.CompilerParams(use_tc_tiling_on_sc=False),
    scratch_types=[
        pltpu.VMEM((B_PER_W,), jnp.int32),
        pltpu.VMEM((2, CH, EMBED), jnp.float32),
        pltpu.VMEM((SEQ, EMBED), jnp.float32),
        pltpu.SemaphoreType.DMA,
        pltpu.SemaphoreType.DMA,
        pltpu.SemaphoreType.DMA,
        pltpu.SemaphoreType.DMA,
    ],
)
def _sc_embed(table_hbm, x_hbm, pe_hbm, out_hbm, *scratch):
    _sc_body(table_hbm, x_hbm, pe_hbm, out_hbm, *scratch)


@jax.jit
def kernel(x, token_table):
    xf = x.reshape(-1).astype(jnp.int32)
    pe = jnp.asarray(_PE)
    out = _sc_embed(token_table, xf, pe)
    return out.reshape(SEQ, BATCH, EMBED)
